# Initial kernel scaffold; baseline (speedup 1.0000x reference)
import jax
import jax.numpy as jnp
from jax.experimental import pallas as pl


def kernel(x, edge_index, edge_weight, W):
    raise NotImplementedError("write your pallas kernel here")


# speedup vs baseline: 2.7063x; 2.7063x over previous
import functools

import jax
import jax.numpy as jnp
from jax import lax
from jax.experimental import pallas as pl
from jax.experimental.pallas import tpu as pltpu
from jax.experimental.pallas import tpu_sc as plsc

NC = 2
NS = 16
L = 16
CH = 128


def _matmul_halves(x, W):
    N, D = x.shape
    H = D // 2
    BN = 400
    assert N % BN == 0

    def mm(x_ref, w_ref, o_ref):
        o_ref[0] = jnp.dot(x_ref[...], w_ref[...],
                           preferred_element_type=jnp.float32)

    return pl.pallas_call(
        mm,
        grid=(N // BN, 2),
        in_specs=[
            pl.BlockSpec((BN, D), lambda i, j: (i, 0)),
            pl.BlockSpec((D, H), lambda i, j: (0, j)),
        ],
        out_specs=pl.BlockSpec((1, BN, H), lambda i, j: (j, i, 0)),
        out_shape=jax.ShapeDtypeStruct((2, N, H), jnp.float32),
    )(x, W)


def _sc_spmm(y0, y1, srcp, dstp, wp, n_acc, tpt):
    N, H = y0.shape
    n_chunks = tpt // CH
    rows_per_tile = n_acc // NS
    zcopies = rows_per_tile // CH
    assert rows_per_tile % CH == 0

    mesh = plsc.VectorSubcoreMesh(core_axis_name="c", subcore_axis_name="s",
                                  num_cores=NC, num_subcores=NS)

    @functools.partial(
        pl.kernel,
        out_type=(jax.ShapeDtypeStruct((n_acc, H), jnp.float32),
                  jax.ShapeDtypeStruct((n_acc, H), jnp.float32)),
        mesh=mesh,
        scratch_types=[
            pltpu.VMEM((CH,), jnp.int32),
            pltpu.VMEM((CH,), jnp.int32),
            pltpu.VMEM((CH,), jnp.float32),
            pltpu.VMEM((CH, H), jnp.float32),
            pltpu.VMEM_SHARED((n_acc, H), jnp.float32),
            pltpu.SemaphoreType.DMA,
        ],
    )
    def k(y0_hbm, y1_hbm, src_hbm, dst_hbm, w_hbm, out0, out1,
          isv, idv, wv, rows, acc, sem):
        c = lax.axis_index("c")
        s = lax.axis_index("s")

        @pl.loop(0, CH)
        def _(i):
            for j in range(H // L):
                rows[i, pl.ds(j * L, L)] = jnp.zeros((L,), jnp.float32)

        @pl.loop(0, zcopies)
        def _(z):
            pltpu.sync_copy(rows, acc.at[pl.ds(s * rows_per_tile + z * CH, CH)])

        plsc.subcore_barrier()

        ebase = s * tpt

        def run(y_hbm):
            @pl.loop(0, n_chunks)
            def _(g):
                off = ebase + g * CH
                pltpu.sync_copy(src_hbm.at[pl.ds(off, CH)], isv)
                pltpu.sync_copy(dst_hbm.at[pl.ds(off, CH)], idv)
                pltpu.sync_copy(w_hbm.at[pl.ds(off, CH)], wv)
                pltpu.async_copy(y_hbm.at[isv], rows, sem).wait()

                @pl.loop(0, CH // L)
                def _(gq):
                    wg = wv[pl.ds(gq * L, L)]
                    for lane in range(L):
                        e = gq * L + lane
                        we = wg[lane]
                        for j in range(H // L):
                            sl = pl.ds(j * L, L)
                            rows[e, sl] = rows[e, sl] * we

                pltpu.sync_copy(rows, acc.at[idv], add=True)

        @pl.when(c == 0)
        def _():
            run(y0_hbm)

        @pl.when(c == 1)
        def _():
            run(y1_hbm)

        plsc.subcore_barrier()

        r0 = s * rows_per_tile

        @pl.when(c == 0)
        def _():
            pltpu.sync_copy(acc.at[pl.ds(r0, rows_per_tile)],
                            out0.at[pl.ds(r0, rows_per_tile)])

        @pl.when(c == 1)
        def _():
            pltpu.sync_copy(acc.at[pl.ds(r0, rows_per_tile)],
                            out1.at[pl.ds(r0, rows_per_tile)])

    return k(y0, y1, srcp, dstp, wp)


def kernel(x, edge_index, edge_weight, W):
    N, D = x.shape
    E = edge_weight.shape[0]

    yh = _matmul_halves(x, W)

    src = edge_index[0].astype(jnp.int32)
    dst = edge_index[1].astype(jnp.int32)
    w = edge_weight.astype(jnp.float32)

    tpt = ((E + NS * CH - 1) // (NS * CH)) * CH
    e_pad = tpt * NS
    n_acc = ((N + NS * CH - 1) // (NS * CH)) * NS * CH

    pad = e_pad - E
    srcp = jnp.concatenate([src, jnp.zeros((pad,), jnp.int32)])
    dstp = jnp.concatenate([dst, jnp.full((pad,), N, jnp.int32)])
    wp = jnp.concatenate([w, jnp.zeros((pad,), jnp.float32)])

    o0, o1 = _sc_spmm(yh[0], yh[1], srcp, dstp, wp, n_acc, tpt)
    return jnp.concatenate([o0[:N], o1[:N]], axis=1)
